# baseline (speedup 1.0000x reference)
import jax
import jax.numpy as jnp
from jax.experimental import pallas as pl
from jax.experimental.pallas import tpu as pltpu

SEQ = 200
DIM = 64
CB = 32
DEPTH = 4


def _body(ex_hbm, cat_hbm, pos_hbm, out_hbm,
          ex_buf, cat_buf, out_buf, pos_vmem,
          ex_sems, cat_sems, out_sems, pos_sem):
    n = ex_hbm.shape[0] // CB

    pos_copy = pltpu.make_async_copy(pos_hbm, pos_vmem, pos_sem)
    pos_copy.start()
    pos_copy.wait()

    def start_in(i, slot):
        pltpu.make_async_copy(
            ex_hbm.at[pl.ds(i * CB, CB)], ex_buf.at[slot], ex_sems.at[slot]
        ).start()
        pltpu.make_async_copy(
            cat_hbm.at[pl.ds(i * CB, CB)], cat_buf.at[slot], cat_sems.at[slot]
        ).start()

    for d in range(DEPTH):
        start_in(d, d)

    for i in range(n):
        slot = i % DEPTH
        pltpu.make_async_copy(
            ex_hbm.at[pl.ds(i * CB, CB)], ex_buf.at[slot], ex_sems.at[slot]
        ).wait()
        pltpu.make_async_copy(
            cat_hbm.at[pl.ds(i * CB, CB)], cat_buf.at[slot], cat_sems.at[slot]
        ).wait()
        if i >= DEPTH:
            pltpu.make_async_copy(
                out_buf.at[slot], out_hbm.at[pl.ds((i - DEPTH) * CB, CB)],
                out_sems.at[slot],
            ).wait()
        out_buf[slot] = ex_buf[slot] + cat_buf[slot] + pos_vmem[:][None]
        pltpu.make_async_copy(
            out_buf.at[slot], out_hbm.at[pl.ds(i * CB, CB)], out_sems.at[slot]
        ).start()
        nxt = i + DEPTH
        if nxt < n:
            start_in(nxt, slot)

    for i in range(n - DEPTH, n):
        slot = i % DEPTH
        pltpu.make_async_copy(
            out_buf.at[slot], out_hbm.at[pl.ds(i * CB, CB)], out_sems.at[slot]
        ).wait()


def kernel(exercises, categories, position_embed):
    B = exercises.shape[0]
    return pl.pallas_call(
        _body,
        in_specs=[
            pl.BlockSpec(memory_space=pl.MemorySpace.ANY),
            pl.BlockSpec(memory_space=pl.MemorySpace.ANY),
            pl.BlockSpec(memory_space=pl.MemorySpace.ANY),
        ],
        out_specs=pl.BlockSpec(memory_space=pl.MemorySpace.ANY),
        out_shape=jax.ShapeDtypeStruct((B, SEQ, DIM), jnp.float32),
        scratch_shapes=[
            pltpu.VMEM((DEPTH, CB, SEQ, DIM), jnp.float32),
            pltpu.VMEM((DEPTH, CB, SEQ, DIM), jnp.float32),
            pltpu.VMEM((DEPTH, CB, SEQ, DIM), jnp.float32),
            pltpu.VMEM((SEQ, DIM), jnp.float32),
            pltpu.SemaphoreType.DMA((DEPTH,)),
            pltpu.SemaphoreType.DMA((DEPTH,)),
            pltpu.SemaphoreType.DMA((DEPTH,)),
            pltpu.SemaphoreType.DMA,
        ],
        compiler_params=pltpu.CompilerParams(vmem_limit_bytes=110 * 1024 * 1024),
    )(exercises, categories, position_embed)


# speedup vs baseline: 6.4094x; 6.4094x over previous
import jax
import jax.numpy as jnp
from jax.experimental import pallas as pl
from jax.experimental.pallas import tpu as pltpu

SEQ = 200
DIM = 64
BS = 8


def _add_kernel(ex_ref, cat_ref, pos_ref, out_ref):
    out_ref[:] = ex_ref[:] + cat_ref[:] + pos_ref[:][:, :, None]


def kernel(exercises, categories, position_embed):
    B = exercises.shape[0]
    ex_t = jnp.transpose(exercises, (1, 2, 0))
    cat_t = jnp.transpose(categories, (1, 2, 0))
    out_t = pl.pallas_call(
        _add_kernel,
        grid=(SEQ // BS,),
        in_specs=[
            pl.BlockSpec((BS, DIM, B), lambda i: (i, 0, 0)),
            pl.BlockSpec((BS, DIM, B), lambda i: (i, 0, 0)),
            pl.BlockSpec((BS, DIM), lambda i: (i, 0)),
        ],
        out_specs=pl.BlockSpec((BS, DIM, B), lambda i: (i, 0, 0)),
        out_shape=jax.ShapeDtypeStruct((SEQ, DIM, B), jnp.float32),
        compiler_params=pltpu.CompilerParams(
            dimension_semantics=("arbitrary",),
        ),
    )(ex_t, cat_t, position_embed)
    return jnp.transpose(out_t, (2, 0, 1))


# speedup vs baseline: 6.4158x; 1.0010x over previous
import jax
import jax.numpy as jnp
from jax.experimental import pallas as pl
from jax.experimental.pallas import tpu as pltpu

SEQ = 200
DIM = 64
BS = 8
BL = 2048


def _add_kernel(ex_ref, cat_ref, pos_ref, out_ref):
    out_ref[:] = ex_ref[:] + cat_ref[:] + pos_ref[:][:, :, None]


def kernel(exercises, categories, position_embed):
    B = exercises.shape[0]
    ex_t = jnp.transpose(exercises, (1, 2, 0))
    cat_t = jnp.transpose(categories, (1, 2, 0))
    out_t = pl.pallas_call(
        _add_kernel,
        grid=(SEQ // BS, B // BL),
        in_specs=[
            pl.BlockSpec((BS, DIM, BL), lambda i, j: (i, 0, j)),
            pl.BlockSpec((BS, DIM, BL), lambda i, j: (i, 0, j)),
            pl.BlockSpec((BS, DIM), lambda i, j: (i, 0)),
        ],
        out_specs=pl.BlockSpec((BS, DIM, BL), lambda i, j: (i, 0, j)),
        out_shape=jax.ShapeDtypeStruct((SEQ, DIM, B), jnp.float32),
        compiler_params=pltpu.CompilerParams(
            dimension_semantics=("arbitrary", "arbitrary"),
        ),
    )(ex_t, cat_t, position_embed)
    return jnp.transpose(out_t, (2, 0, 1))
